# Initial kernel scaffold; baseline (speedup 1.0000x reference)
import jax
import jax.numpy as jnp
from jax.experimental import pallas as pl


def kernel(feat, edge_index, W_self, W_neigh):
    raise NotImplementedError("write your pallas kernel here")


# speedup vs baseline: 5.3871x; 5.3871x over previous
import jax
import jax.numpy as jnp
from jax import lax
from jax.experimental import pallas as pl
from jax.experimental.pallas import tpu as pltpu
from jax.experimental.pallas import tpu_sc as plsc

N_NODES_ = 10000
D_ = 128
DH = 64
K_ = 32
N_EDGES_ = 320000

NW = 32
CHUNK = 128
CHUNKS_PER_TILE = 79
EDGES_PAD = NW * CHUNKS_PER_TILE * CHUNK
ROWS = 10240
DUMP_ROW = N_NODES_
ROWS_PER_TILE = ROWS // 16
WB_CHUNK = 128
BR = 512


def _matmul_maxk_body(feat_ref, ws_ref, wn_ref, hself_ref, h0_ref, h1_ref):
    f = feat_ref[...]
    hs = jnp.dot(f, ws_ref[...], preferred_element_type=jnp.float32)
    hn = jnp.dot(f, wn_ref[...], preferred_element_type=jnp.float32)
    hself_ref[...] = hs

    cols = lax.broadcasted_iota(jnp.int32, hn.shape, 1)
    work = hn
    keep = jnp.zeros(hn.shape, dtype=jnp.bool_)
    for _ in range(K_):
        m = jnp.max(work, axis=1, keepdims=True)
        is_max = work == m
        cand = jnp.where(is_max, cols, D_)
        j = jnp.min(cand, axis=1, keepdims=True)
        sel = cols == j
        keep = jnp.logical_or(keep, sel)
        work = jnp.where(sel, -jnp.inf, work)
    hsp = jnp.where(keep, hn, 0.0)
    h0_ref[...] = hsp[:, :DH]
    h1_ref[...] = hsp[:, DH:]


def _matmul_maxk(feat_p, w_self, w_neigh):
    return pl.pallas_call(
        _matmul_maxk_body,
        grid=(ROWS // BR,),
        in_specs=[
            pl.BlockSpec((BR, D_), lambda i: (i, 0)),
            pl.BlockSpec((D_, D_), lambda i: (0, 0)),
            pl.BlockSpec((D_, D_), lambda i: (0, 0)),
        ],
        out_specs=[
            pl.BlockSpec((BR, D_), lambda i: (i, 0)),
            pl.BlockSpec((BR, DH), lambda i: (i, 0)),
            pl.BlockSpec((BR, DH), lambda i: (i, 0)),
        ],
        out_shape=[
            jax.ShapeDtypeStruct((ROWS, D_), jnp.float32),
            jax.ShapeDtypeStruct((ROWS, DH), jnp.float32),
            jax.ShapeDtypeStruct((ROWS, DH), jnp.float32),
        ],
        compiler_params=pltpu.CompilerParams(
            dimension_semantics=("arbitrary",),
        ),
    )(feat_p, w_self, w_neigh)


def _sc_aggregate_body(h0_hbm, h1_hbm, src_hbm, dst_hbm, agg_hbm, deg_hbm,
                       src_v, dst_v, rows_v, deg_v, acc_s, gsem):
    c = lax.axis_index("c")
    s = lax.axis_index("s")
    wid = c * 16 + s
    base = s * ROWS_PER_TILE

    pltpu.sync_copy(src_hbm.at[wid], src_v)
    pltpu.sync_copy(dst_hbm.at[wid], dst_v)

    def _zero_deg(i, _):
        deg_v[pl.ds(i * 16, 16)] = jnp.zeros((16,), jnp.float32)
        return 0

    lax.fori_loop(0, ROWS // 16, _zero_deg, 0)

    ones16 = jnp.ones((16,), jnp.float32)

    for p in range(2):
        h_hbm = h0_hbm if p == 0 else h1_hbm

        def _fill_zero(i, _):
            for j in range(DH // 16):
                rows_v[i, pl.ds(j * 16, 16)] = jnp.zeros((16,), jnp.float32)
            return 0

        lax.fori_loop(0, CHUNK, _fill_zero, 0)

        def _zero_acc(i, _):
            pltpu.sync_copy(rows_v, acc_s.at[pl.ds(base + i * WB_CHUNK, WB_CHUNK)])
            return 0

        lax.fori_loop(0, ROWS_PER_TILE // WB_CHUNK, _zero_acc, 0)

        plsc.subcore_barrier()

        def _edge_chunk(j, _):
            src_row = src_v.at[j]
            dst_row = dst_v.at[j]
            pltpu.async_copy(h_hbm.at[src_row], rows_v, gsem).wait()
            pltpu.sync_copy(rows_v, acc_s.at[dst_row], add=True)
            if p == 0:
                for g in range(CHUNK // 16):
                    idx16 = dst_v[j, pl.ds(g * 16, 16)]
                    plsc.addupdate_scatter(deg_v, [idx16], ones16)
            return 0

        lax.fori_loop(0, CHUNKS_PER_TILE, _edge_chunk, 0)

        plsc.subcore_barrier()

        def _writeback(i, _):
            r = base + i * WB_CHUNK
            pltpu.sync_copy(acc_s.at[pl.ds(r, WB_CHUNK)], rows_v)
            pltpu.sync_copy(rows_v, agg_hbm.at[c].at[p].at[pl.ds(r, WB_CHUNK)])
            return 0

        lax.fori_loop(0, ROWS_PER_TILE // WB_CHUNK, _writeback, 0)

    pltpu.sync_copy(deg_v, deg_hbm.at[wid])


def _sc_aggregate(h0, h1, src3d, dst3d):
    mesh = plsc.VectorSubcoreMesh(core_axis_name="c", subcore_axis_name="s")
    kern = pl.kernel(
        _sc_aggregate_body,
        mesh=mesh,
        out_type=[
            jax.ShapeDtypeStruct((2, 2, ROWS, DH), jnp.float32),
            jax.ShapeDtypeStruct((NW, ROWS), jnp.float32),
        ],
        scratch_types=[
            pltpu.VMEM((CHUNKS_PER_TILE, CHUNK), jnp.int32),
            pltpu.VMEM((CHUNKS_PER_TILE, CHUNK), jnp.int32),
            pltpu.VMEM((CHUNK, DH), jnp.float32),
            pltpu.VMEM((ROWS,), jnp.float32),
            pltpu.VMEM_SHARED((ROWS, DH), jnp.float32),
            pltpu.SemaphoreType.DMA,
        ],
        compiler_params=pltpu.CompilerParams(
            needs_layout_passes=False,
            use_tc_tiling_on_sc=False,
        ),
    )
    return kern(h0, h1, src3d, dst3d)


def _combine_body(hself_ref, agg_ref, deg_ref, out_ref):
    a_lo = agg_ref[0, 0] + agg_ref[1, 0]
    a_hi = agg_ref[0, 1] + agg_ref[1, 1]
    a = jnp.concatenate([a_lo, a_hi], axis=1)
    d = lax.dot_general(
        deg_ref[...], jnp.ones((NW, 1), jnp.float32),
        (((0,), (0,)), ((), ())),
        preferred_element_type=jnp.float32,
    )
    out_ref[...] = hself_ref[...] + a / jnp.maximum(d, 1.0)


def _combine(h_self, agg, deg):
    return pl.pallas_call(
        _combine_body,
        grid=(ROWS // BR,),
        in_specs=[
            pl.BlockSpec((BR, D_), lambda i: (i, 0)),
            pl.BlockSpec((2, 2, BR, DH), lambda i: (0, 0, i, 0)),
            pl.BlockSpec((NW, BR), lambda i: (0, i)),
        ],
        out_specs=pl.BlockSpec((BR, D_), lambda i: (i, 0)),
        out_shape=jax.ShapeDtypeStruct((ROWS, D_), jnp.float32),
        compiler_params=pltpu.CompilerParams(
            dimension_semantics=("arbitrary",),
        ),
    )(h_self, agg, deg)


def kernel(feat, edge_index, W_self, W_neigh):
    feat_p = jnp.pad(feat, ((0, ROWS - N_NODES_), (0, 0)))
    h_self, h0, h1 = _matmul_maxk(feat_p, W_self, W_neigh)

    src = edge_index[0]
    dst = edge_index[1]
    pad = EDGES_PAD - N_EDGES_
    src_p = jnp.concatenate([src, jnp.zeros((pad,), jnp.int32)])
    dst_p = jnp.concatenate([dst, jnp.full((pad,), DUMP_ROW, jnp.int32)])
    src3d = src_p.reshape(NW, CHUNKS_PER_TILE, CHUNK)
    dst3d = dst_p.reshape(NW, CHUNKS_PER_TILE, CHUNK)

    agg, deg = _sc_aggregate(h0, h1, src3d, dst3d)
    out = _combine(h_self, agg, deg)
    return out[:N_NODES_]
